# baseline (speedup 1.0000x reference)
import functools

import jax
import jax.numpy as jnp
from jax.experimental import pallas as pl
from jax.experimental.pallas import tpu as pltpu


def _round_up(n, m):
    return ((n + m - 1) // m) * m


def _fused_ffn_ln(x_ref, w1_ref, b1_ref, w2_ref, v_ref, o_ref, *,
                  eps, inv_d, n_chunks, chunk):
    xb = x_ref[...]
    x32 = xb.astype(jnp.float32)
    y = x32 + v_ref[0, :]
    for k in range(n_chunks):
        sl = pl.ds(k * chunk, chunk)
        h = jnp.dot(xb, w1_ref[:, sl], preferred_element_type=jnp.float32)
        h = jnp.maximum(h + b1_ref[:, sl], 0.0)
        y = y + jnp.dot(h.astype(jnp.bfloat16), w2_ref[sl, :],
                        preferred_element_type=jnp.float32)
    mean = jnp.sum(y, axis=-1, keepdims=True) * inv_d
    var = jnp.sum(y * y, axis=-1, keepdims=True) * inv_d - mean * mean
    var = jnp.maximum(var, 0.0)
    o_ref[...] = (y - mean) * jax.lax.rsqrt(var + eps) * v_ref[1, :] + v_ref[2, :]


def _ffn_pallas(x2, w1b, b1r, w2b, vecs, *, tile_rows, chunk, eps, inv_d,
                out_dtype):
    N_p, d_in_p = x2.shape
    d_hid_p = w1b.shape[1]
    n_row_tiles = N_p // tile_rows
    n_chunks = d_hid_p // chunk

    weight_bytes = (w1b.size + w2b.size) * 2 + (b1r.size + vecs.size) * 4
    cost = pl.CostEstimate(
        flops=4 * N_p * d_in_p * d_hid_p,
        transcendentals=N_p,
        bytes_accessed=N_p * d_in_p * 6 + weight_bytes,
    )

    return pl.pallas_call(
        functools.partial(_fused_ffn_ln, eps=eps, inv_d=inv_d,
                          n_chunks=n_chunks, chunk=chunk),
        out_shape=jax.ShapeDtypeStruct((N_p, d_in_p), out_dtype),
        grid=(n_row_tiles,),
        in_specs=[
            pl.BlockSpec((tile_rows, d_in_p), lambda i: (i, 0)),
            pl.BlockSpec((d_in_p, d_hid_p), lambda i: (0, 0)),
            pl.BlockSpec((1, d_hid_p), lambda i: (0, 0)),
            pl.BlockSpec((d_hid_p, d_in_p), lambda i: (0, 0)),
            pl.BlockSpec((8, d_in_p), lambda i: (0, 0)),
        ],
        out_specs=pl.BlockSpec((tile_rows, d_in_p), lambda i: (i, 0)),
        compiler_params=pltpu.CompilerParams(
            dimension_semantics=("parallel",),
            vmem_limit_bytes=int((64 << 20) * 0.98),
        ),
        cost_estimate=cost,
    )(x2, w1b, b1r, w2b, vecs)


def kernel(x, w1, b1, w2, b2, gamma, beta, *, eps=1e-6, tile_rows=256,
           chunk=1024):
    B, S, d_in = x.shape
    d_hid = w1.shape[1]
    N = B * S

    d_in_p = _round_up(d_in, 128)
    d_hid_p = _round_up(max(d_hid, chunk), chunk)
    N_p = _round_up(N, tile_rows)

    x2 = x.reshape(N, d_in)
    if N_p != N or d_in_p != d_in:
        x2 = jnp.pad(x2, ((0, N_p - N), (0, d_in_p - d_in)))
    if d_in_p != d_in or d_hid_p != d_hid:
        w1 = jnp.pad(w1, ((0, d_in_p - d_in), (0, d_hid_p - d_hid)))
        w2 = jnp.pad(w2, ((0, d_hid_p - d_hid), (0, d_in_p - d_in)))
        b1 = jnp.pad(b1, (0, d_hid_p - d_hid))
        b2 = jnp.pad(b2, (0, d_in_p - d_in))
        gamma = jnp.pad(gamma, (0, d_in_p - d_in))
        beta = jnp.pad(beta, (0, d_in_p - d_in))

    xb16 = x2.astype(jnp.bfloat16)
    w1b = w1.astype(jnp.bfloat16)
    w2b = w2.astype(jnp.bfloat16)
    b1r = b1.reshape(1, d_hid_p)
    vecs = jnp.zeros((8, d_in_p), jnp.float32)
    vecs = vecs.at[0].set(b2).at[1].set(gamma).at[2].set(beta)

    out = _ffn_pallas(xb16, w1b, b1r, w2b, vecs, tile_rows=tile_rows,
                      chunk=chunk, eps=eps, inv_d=1.0 / d_in,
                      out_dtype=x.dtype)

    if N_p != N or d_in_p != d_in:
        out = out[:N, :d_in]
    return out.reshape(B, S, d_in)


# speedup vs baseline: 1.0931x; 1.0931x over previous
import functools

import jax
import jax.numpy as jnp
from jax.experimental import pallas as pl
from jax.experimental.pallas import tpu as pltpu


def _round_up(n, m):
    return ((n + m - 1) // m) * m


def _fused_ffn_ln(x_ref, w1_ref, b1_ref, w2_ref, v_ref, o_ref, *,
                  eps, inv_d, n_chunks, chunk):
    x32 = x_ref[...]
    xb = x32.astype(jnp.bfloat16)
    y = x32 + v_ref[0, :]
    for k in range(n_chunks):
        sl = pl.ds(k * chunk, chunk)
        h = jnp.dot(xb, w1_ref[:, sl], preferred_element_type=jnp.float32)
        h = jnp.maximum(h + b1_ref[:, sl], 0.0)
        y = y + jnp.dot(h.astype(jnp.bfloat16), w2_ref[sl, :],
                        preferred_element_type=jnp.float32)
    mean = jnp.sum(y, axis=-1, keepdims=True) * inv_d
    var = jnp.sum(y * y, axis=-1, keepdims=True) * inv_d - mean * mean
    var = jnp.maximum(var, 0.0)
    o_ref[...] = (y - mean) * jax.lax.rsqrt(var + eps) * v_ref[1, :] + v_ref[2, :]


def _ffn_pallas(x2, w1b, b1r, w2b, vecs, *, tile_rows, chunk, eps, inv_d,
                out_dtype):
    N_p, d_in_p = x2.shape
    d_hid_p = w1b.shape[1]
    n_row_tiles = N_p // tile_rows
    n_chunks = d_hid_p // chunk

    weight_bytes = (w1b.size + w2b.size) * 2 + (b1r.size + vecs.size) * 4
    cost = pl.CostEstimate(
        flops=4 * N_p * d_in_p * d_hid_p,
        transcendentals=N_p,
        bytes_accessed=N_p * d_in_p * 6 + weight_bytes,
    )

    return pl.pallas_call(
        functools.partial(_fused_ffn_ln, eps=eps, inv_d=inv_d,
                          n_chunks=n_chunks, chunk=chunk),
        out_shape=jax.ShapeDtypeStruct((N_p, d_in_p), out_dtype),
        grid=(n_row_tiles,),
        in_specs=[
            pl.BlockSpec((tile_rows, d_in_p), lambda i: (i, 0)),
            pl.BlockSpec((d_in_p, d_hid_p), lambda i: (0, 0)),
            pl.BlockSpec((1, d_hid_p), lambda i: (0, 0)),
            pl.BlockSpec((d_hid_p, d_in_p), lambda i: (0, 0)),
            pl.BlockSpec((8, d_in_p), lambda i: (0, 0)),
        ],
        out_specs=pl.BlockSpec((tile_rows, d_in_p), lambda i: (i, 0)),
        compiler_params=pltpu.CompilerParams(
            dimension_semantics=("parallel",),
            vmem_limit_bytes=int((64 << 20) * 0.98),
        ),
        cost_estimate=cost,
    )(x2, w1b, b1r, w2b, vecs)


def kernel(x, w1, b1, w2, b2, gamma, beta, *, eps=1e-6, tile_rows=256,
           chunk=4096):
    B, S, d_in = x.shape
    d_hid = w1.shape[1]
    N = B * S

    d_in_p = _round_up(d_in, 128)
    d_hid_p = _round_up(max(d_hid, chunk), chunk)
    N_p = _round_up(N, tile_rows)

    x2 = x.reshape(N, d_in)
    if N_p != N or d_in_p != d_in:
        x2 = jnp.pad(x2, ((0, N_p - N), (0, d_in_p - d_in)))
    if d_in_p != d_in or d_hid_p != d_hid:
        w1 = jnp.pad(w1, ((0, d_in_p - d_in), (0, d_hid_p - d_hid)))
        w2 = jnp.pad(w2, ((0, d_hid_p - d_hid), (0, d_in_p - d_in)))
        b1 = jnp.pad(b1, (0, d_hid_p - d_hid))
        b2 = jnp.pad(b2, (0, d_in_p - d_in))
        gamma = jnp.pad(gamma, (0, d_in_p - d_in))
        beta = jnp.pad(beta, (0, d_in_p - d_in))

    w1b = w1.astype(jnp.bfloat16)
    w2b = w2.astype(jnp.bfloat16)
    b1r = b1.reshape(1, d_hid_p)
    vecs = jnp.zeros((8, d_in_p), jnp.float32)
    vecs = vecs.at[0].set(b2).at[1].set(gamma).at[2].set(beta)

    out = _ffn_pallas(x2, w1b, b1r, w2b, vecs, tile_rows=tile_rows,
                      chunk=chunk, eps=eps, inv_d=1.0 / d_in,
                      out_dtype=x.dtype)

    if N_p != N or d_in_p != d_in:
        out = out[:N, :d_in]
    return out.reshape(B, S, d_in)


# speedup vs baseline: 1.1237x; 1.0280x over previous
import functools

import jax
import jax.numpy as jnp
from jax.experimental import pallas as pl
from jax.experimental.pallas import tpu as pltpu


def _round_up(n, m):
    return ((n + m - 1) // m) * m


def _fused_ffn_ln(x_ref, w1_ref, b1_ref, w2_ref, v_ref, o_ref, *,
                  eps, inv_d, n_chunks, chunk):
    x32 = x_ref[...]
    xb = x32
    y = x32 + v_ref[0, :]
    for k in range(n_chunks):
        sl = pl.ds(k * chunk, chunk)
        h = jnp.dot(xb, w1_ref[:, sl], preferred_element_type=jnp.float32)
        h = jnp.maximum(h + b1_ref[:, sl], 0.0)
        y = y + jnp.dot(h, w2_ref[sl, :],
                        preferred_element_type=jnp.float32)
    mean = jnp.sum(y, axis=-1, keepdims=True) * inv_d
    var = jnp.sum(y * y, axis=-1, keepdims=True) * inv_d - mean * mean
    var = jnp.maximum(var, 0.0)
    o_ref[...] = (y - mean) * jax.lax.rsqrt(var + eps) * v_ref[1, :] + v_ref[2, :]


def _ffn_pallas(x2, w1b, b1r, w2b, vecs, *, tile_rows, chunk, eps, inv_d,
                out_dtype):
    N_p, d_in_p = x2.shape
    d_hid_p = w1b.shape[1]
    n_row_tiles = N_p // tile_rows
    n_chunks = d_hid_p // chunk

    weight_bytes = (w1b.size + w2b.size) * 2 + (b1r.size + vecs.size) * 4
    cost = pl.CostEstimate(
        flops=4 * N_p * d_in_p * d_hid_p,
        transcendentals=N_p,
        bytes_accessed=N_p * d_in_p * 6 + weight_bytes,
    )

    return pl.pallas_call(
        functools.partial(_fused_ffn_ln, eps=eps, inv_d=inv_d,
                          n_chunks=n_chunks, chunk=chunk),
        out_shape=jax.ShapeDtypeStruct((N_p, d_in_p), out_dtype),
        grid=(n_row_tiles,),
        in_specs=[
            pl.BlockSpec((tile_rows, d_in_p), lambda i: (i, 0)),
            pl.BlockSpec((d_in_p, d_hid_p), lambda i: (0, 0)),
            pl.BlockSpec((1, d_hid_p), lambda i: (0, 0)),
            pl.BlockSpec((d_hid_p, d_in_p), lambda i: (0, 0)),
            pl.BlockSpec((8, d_in_p), lambda i: (0, 0)),
        ],
        out_specs=pl.BlockSpec((tile_rows, d_in_p), lambda i: (i, 0)),
        compiler_params=pltpu.CompilerParams(
            dimension_semantics=("parallel",),
            vmem_limit_bytes=int((64 << 20) * 0.98),
        ),
        cost_estimate=cost,
    )(x2, w1b, b1r, w2b, vecs)


def kernel(x, w1, b1, w2, b2, gamma, beta, *, eps=1e-6, tile_rows=256,
           chunk=4096):
    B, S, d_in = x.shape
    d_hid = w1.shape[1]
    N = B * S

    d_in_p = _round_up(d_in, 128)
    d_hid_p = _round_up(max(d_hid, chunk), chunk)
    N_p = _round_up(N, tile_rows)

    x2 = x.reshape(N, d_in)
    if N_p != N or d_in_p != d_in:
        x2 = jnp.pad(x2, ((0, N_p - N), (0, d_in_p - d_in)))
    if d_in_p != d_in or d_hid_p != d_hid:
        w1 = jnp.pad(w1, ((0, d_in_p - d_in), (0, d_hid_p - d_hid)))
        w2 = jnp.pad(w2, ((0, d_hid_p - d_hid), (0, d_in_p - d_in)))
        b1 = jnp.pad(b1, (0, d_hid_p - d_hid))
        b2 = jnp.pad(b2, (0, d_in_p - d_in))
        gamma = jnp.pad(gamma, (0, d_in_p - d_in))
        beta = jnp.pad(beta, (0, d_in_p - d_in))

    w1b = w1
    w2b = w2
    b1r = b1.reshape(1, d_hid_p)
    vecs = jnp.zeros((8, d_in_p), jnp.float32)
    vecs = vecs.at[0].set(b2).at[1].set(gamma).at[2].set(beta)

    out = _ffn_pallas(x2, w1b, b1r, w2b, vecs, tile_rows=tile_rows,
                      chunk=chunk, eps=eps, inv_d=1.0 / d_in,
                      out_dtype=x.dtype)

    if N_p != N or d_in_p != d_in:
        out = out[:N, :d_in]
    return out.reshape(B, S, d_in)


# speedup vs baseline: 1.1560x; 1.0287x over previous
import functools

import jax
import jax.numpy as jnp
from jax.experimental import pallas as pl
from jax.experimental.pallas import tpu as pltpu


def _round_up(n, m):
    return ((n + m - 1) // m) * m


def _ffn_ln_chain(x32, w1_ref, b1_ref, w2_ref, v_ref, *, eps, inv_d):
    h = jnp.dot(x32, w1_ref[...], preferred_element_type=jnp.float32)
    h = jnp.maximum(h + b1_ref[...], 0.0)
    y = jnp.dot(h, w2_ref[...], preferred_element_type=jnp.float32)
    y = y + v_ref[0, :] + x32
    mean = jnp.sum(y, axis=-1, keepdims=True) * inv_d
    var = jnp.sum(y * y, axis=-1, keepdims=True) * inv_d - mean * mean
    var = jnp.maximum(var, 0.0)
    return (y - mean) * jax.lax.rsqrt(var + eps) * v_ref[1, :] + v_ref[2, :]


def _ffn_kernel(x_ref, w1_ref, b1_ref, w2_ref, v_ref, o_ref, *,
                eps, inv_d, n_sub, sub_rows):
    for s in range(n_sub):
        rows = slice(s * sub_rows, (s + 1) * sub_rows)
        o_ref[rows, :] = _ffn_ln_chain(x_ref[rows, :], w1_ref, b1_ref,
                                       w2_ref, v_ref, eps=eps, inv_d=inv_d)


def _ffn_pallas(x2, w1p, b1r, w2p, vecs, *, tile_rows, sub_rows, eps, inv_d,
                out_dtype):
    N_p, d_in_p = x2.shape
    d_hid_p = w1p.shape[1]
    n_tiles = N_p // tile_rows

    weight_bytes = (w1p.size + w2p.size + b1r.size + vecs.size) * 4
    cost = pl.CostEstimate(
        flops=4 * N_p * d_in_p * d_hid_p,
        transcendentals=N_p,
        bytes_accessed=N_p * d_in_p * 8 + weight_bytes,
    )

    return pl.pallas_call(
        functools.partial(_ffn_kernel, eps=eps, inv_d=inv_d,
                          n_sub=tile_rows // sub_rows, sub_rows=sub_rows),
        out_shape=jax.ShapeDtypeStruct((N_p, d_in_p), out_dtype),
        grid=(n_tiles,),
        in_specs=[
            pl.BlockSpec((tile_rows, d_in_p), lambda i: (i, 0)),
            pl.BlockSpec((d_in_p, d_hid_p), lambda i: (0, 0)),
            pl.BlockSpec((1, d_hid_p), lambda i: (0, 0)),
            pl.BlockSpec((d_hid_p, d_in_p), lambda i: (0, 0)),
            pl.BlockSpec((8, d_in_p), lambda i: (0, 0)),
        ],
        out_specs=pl.BlockSpec((tile_rows, d_in_p), lambda i: (i, 0)),
        compiler_params=pltpu.CompilerParams(
            dimension_semantics=("parallel",),
            vmem_limit_bytes=int((64 << 20) * 0.98),
        ),
        cost_estimate=cost,
    )(x2, w1p, b1r, w2p, vecs)


def kernel(x, w1, b1, w2, b2, gamma, beta, *, eps=1e-6, tile_rows=512,
           sub_rows=256):
    B, S, d_in = x.shape
    d_hid = w1.shape[1]
    N = B * S

    d_in_p = _round_up(d_in, 128)
    d_hid_p = _round_up(d_hid, 128)
    N_p = _round_up(N, tile_rows)

    x2 = x.reshape(N, d_in)
    if N_p != N or d_in_p != d_in:
        x2 = jnp.pad(x2, ((0, N_p - N), (0, d_in_p - d_in)))
    if d_in_p != d_in or d_hid_p != d_hid:
        w1 = jnp.pad(w1, ((0, d_in_p - d_in), (0, d_hid_p - d_hid)))
        w2 = jnp.pad(w2, ((0, d_hid_p - d_hid), (0, d_in_p - d_in)))
        b1 = jnp.pad(b1, (0, d_hid_p - d_hid))
        b2 = jnp.pad(b2, (0, d_in_p - d_in))
        gamma = jnp.pad(gamma, (0, d_in_p - d_in))
        beta = jnp.pad(beta, (0, d_in_p - d_in))

    b1r = b1.reshape(1, d_hid_p)
    vecs = jnp.zeros((8, d_in_p), jnp.float32)
    vecs = vecs.at[0].set(b2).at[1].set(gamma).at[2].set(beta)

    out = _ffn_pallas(x2, w1, b1r, w2, vecs, tile_rows=tile_rows,
                      sub_rows=sub_rows, eps=eps, inv_d=1.0 / d_in,
                      out_dtype=x.dtype)

    if N_p != N or d_in_p != d_in:
        out = out[:N, :d_in]
    return out.reshape(B, S, d_in)
